# baseline (speedup 1.0000x reference)
import functools

import jax
import jax.numpy as jnp
from jax import lax
from jax.experimental import pallas as pl
from jax.experimental.pallas import tpu as pltpu
from jax.experimental.pallas import tpu_sc as plsc

N = 2048
E = 32768
D_IN = 512
D_HID = 256
BLK = 256
LANES = 16
NW = 32
ROWS = 32
PASSES = 2
UNROLL = 8
ZUNROLL = 8


def _build_adj_body(idx_hbm, a_hbm, blk, idxv):
    cid = lax.axis_index("c")
    sid = lax.axis_index("s")
    wid = sid * 2 + cid
    ones = jnp.full((LANES,), 1.0, jnp.float32)
    zv = jnp.zeros((LANES,), jnp.float32)
    pltpu.sync_copy(idx_hbm, idxv)
    for p in range(PASSES):
        base = (p * (N // PASSES) + wid * ROWS) * N

        def zbody(i, carry):
            for u in range(ZUNROLL):
                blk[pl.ds((i * ZUNROLL + u) * LANES, LANES)] = zv
            return carry

        lax.fori_loop(0, ROWS * N // LANES // ZUNROLL, zbody, 0)

        def body(i, carry):
            for u in range(UNROLL):
                g = i * UNROLL + u
                d = idxv[pl.ds(g * LANES, LANES)] - base
                du = lax.bitcast_convert_type(d, jnp.uint32)
                m = du < jnp.uint32(ROWS * N)
                plsc.addupdate_scatter(blk, [d], ones, mask=m)
            return carry

        lax.fori_loop(0, E // LANES // UNROLL, body, 0)
        pltpu.sync_copy(blk, a_hbm.at[pl.ds(base, ROWS * N)])


@jax.jit
def _build_adj(idx):
    a_flat = pl.kernel(
        _build_adj_body,
        out_type=jax.ShapeDtypeStruct((N * N,), jnp.float32),
        mesh=plsc.VectorSubcoreMesh(core_axis_name="c", subcore_axis_name="s"),
        compiler_params=pltpu.CompilerParams(needs_layout_passes=False),
        scratch_types=[
            pltpu.VMEM((ROWS * N,), jnp.float32),
            pltpu.VMEM((E,), jnp.int32),
        ],
    )(idx)
    return a_flat.reshape(N, N)


def _scales_body(a_ref, x_ref, z1_ref, n2_ref, n3_ref):
    deg = jnp.maximum(jnp.sum(a_ref[...], axis=1, keepdims=True), 1.0)
    rin = 1.0 / deg
    n1 = lax.rsqrt(deg)
    z1_ref[...] = x_ref[...] * n1
    n2_ref[...] = rin * rin
    n3_ref[...] = n1 * rin


def _wz_body(ar_ref, ac_ref, d2_ref, z1_ref, w_ref, z2_ref):
    i = pl.program_id(0)
    q = ac_ref[...] * d2_ref[...]
    acc = jnp.dot(ar_ref[...], q, preferred_element_type=jnp.float32)
    ri = lax.broadcasted_iota(jnp.int32, (BLK, BLK), 0)
    ci = lax.broadcasted_iota(jnp.int32, (BLK, BLK), 1)
    diag = jnp.sum(jnp.where(ri == ci, acc, 0.0), axis=1, keepdims=True)
    d2_blk = d2_ref[pl.ds(i * BLK, BLK), :]
    w_ref[...] = diag * d2_blk
    t = jnp.dot(ar_ref[...], z1_ref[...], preferred_element_type=jnp.float32)
    z2_ref[...] = t * d2_blk


def _final_body(ar_ref, z2_ref, n3_ref, w_ref, x_ref, wnt_ref, wgt_ref,
                bn_ref, bg_ref, out_ref):
    t2 = jnp.dot(ar_ref[...], z2_ref[...], preferred_element_type=jnp.float32)
    agg = t2 * n3_ref[...]
    xb = x_ref[...]
    eg = agg - xb * w_ref[...]
    a = jnp.dot(xb, wnt_ref[...], preferred_element_type=jnp.float32) + bn_ref[...]
    b = jnp.dot(eg, wgt_ref[...], preferred_element_type=jnp.float32) + bg_ref[...]
    eps = 1e-8
    an = jnp.maximum(jnp.sqrt(jnp.sum(a * a, axis=1, keepdims=True)), eps)
    bn = jnp.maximum(jnp.sqrt(jnp.sum(b * b, axis=1, keepdims=True)), eps)
    out_ref[...] = -jnp.sum(a * b, axis=1, keepdims=True) / (an * bn)


def _col(shape_rows):
    return pl.BlockSpec((shape_rows, 1), lambda i: (i, 0))


def kernel(x, edge_index, fc_n_w, fc_n_b, fc_g_w, fc_g_b):
    ei0 = edge_index[0].astype(jnp.int32)
    ei1 = edge_index[1].astype(jnp.int32)
    idx = ei0 * N + ei1

    a_mat = _build_adj(idx)

    grid = N // BLK
    row_blk = pl.BlockSpec((BLK, N), lambda i: (i, 0))
    full_col = pl.BlockSpec((N, 1), lambda i: (0, 0))

    z1, n2c, n3c = pl.pallas_call(
        _scales_body,
        grid=(grid,),
        compiler_params=pltpu.CompilerParams(dimension_semantics=("parallel",)),
        in_specs=[row_blk, pl.BlockSpec((BLK, D_IN), lambda i: (i, 0))],
        out_specs=[pl.BlockSpec((BLK, D_IN), lambda i: (i, 0)), _col(BLK), _col(BLK)],
        out_shape=[
            jax.ShapeDtypeStruct((N, D_IN), jnp.float32),
            jax.ShapeDtypeStruct((N, 1), jnp.float32),
            jax.ShapeDtypeStruct((N, 1), jnp.float32),
        ],
    )(a_mat, x)

    w, z2 = pl.pallas_call(
        _wz_body,
        grid=(grid,),
        compiler_params=pltpu.CompilerParams(dimension_semantics=("parallel",)),
        in_specs=[
            row_blk,
            pl.BlockSpec((N, BLK), lambda i: (0, i)),
            full_col,
            pl.BlockSpec((N, D_IN), lambda i: (0, 0)),
        ],
        out_specs=[_col(BLK), pl.BlockSpec((BLK, D_IN), lambda i: (i, 0))],
        out_shape=[
            jax.ShapeDtypeStruct((N, 1), jnp.float32),
            jax.ShapeDtypeStruct((N, D_IN), jnp.float32),
        ],
    )(a_mat, a_mat, n2c, z1)

    s = pl.pallas_call(
        _final_body,
        grid=(grid,),
        compiler_params=pltpu.CompilerParams(dimension_semantics=("parallel",)),
        in_specs=[
            row_blk,
            pl.BlockSpec((N, D_IN), lambda i: (0, 0)),
            _col(BLK),
            _col(BLK),
            pl.BlockSpec((BLK, D_IN), lambda i: (i, 0)),
            pl.BlockSpec((D_IN, D_HID), lambda i: (0, 0)),
            pl.BlockSpec((D_IN, D_HID), lambda i: (0, 0)),
            pl.BlockSpec((1, D_HID), lambda i: (0, 0)),
            pl.BlockSpec((1, D_HID), lambda i: (0, 0)),
        ],
        out_specs=_col(BLK),
        out_shape=jax.ShapeDtypeStruct((N, 1), jnp.float32),
    )(a_mat, z2, n3c, w, x, fc_n_w.T, fc_g_w.T,
      fc_n_b.reshape(1, D_HID), fc_g_b.reshape(1, D_HID))

    return s.reshape(1, N)


# speedup vs baseline: 1.0702x; 1.0702x over previous
import jax
import jax.numpy as jnp
from jax import lax
from jax.experimental import pallas as pl
from jax.experimental.pallas import tpu as pltpu
from jax.experimental.pallas import tpu_sc as plsc

N = 2048
E = 32768
D_IN = 512
D_HID = 256
BLK = 256
LANES = 16
NW = 32
ROWS = 32
PASSES = 2
UNROLL = 16
ZUNROLL = 8


def _build_adj_body(idx_hbm, a_hbm, blk, idxv):
    cid = lax.axis_index("c")
    sid = lax.axis_index("s")
    wid = sid * 2 + cid
    ones = jnp.full((LANES,), 1.0, jnp.float32)
    zv = jnp.zeros((LANES,), jnp.float32)
    pltpu.sync_copy(idx_hbm, idxv)
    for p in range(PASSES):
        base = (p * (N // PASSES) + wid * ROWS) * N

        def zbody(i, carry):
            for u in range(ZUNROLL):
                blk[pl.ds((i * ZUNROLL + u) * LANES, LANES)] = zv
            return carry

        lax.fori_loop(0, ROWS * N // LANES // ZUNROLL, zbody, 0)

        def body(i, carry):
            for u in range(UNROLL):
                g = i * UNROLL + u
                d = idxv[pl.ds(g * LANES, LANES)] - base
                du = lax.bitcast_convert_type(d, jnp.uint32)
                m = du < jnp.uint32(ROWS * N)
                plsc.addupdate_scatter(blk, [d], ones, mask=m)
            return carry

        lax.fori_loop(0, E // LANES // UNROLL, body, 0)
        pltpu.sync_copy(blk, a_hbm.at[pl.ds(base, ROWS * N)])


@jax.jit
def _build_adj(idx):
    a_flat = pl.kernel(
        _build_adj_body,
        out_type=jax.ShapeDtypeStruct((N * N,), jnp.float32),
        mesh=plsc.VectorSubcoreMesh(core_axis_name="c", subcore_axis_name="s"),
        compiler_params=pltpu.CompilerParams(needs_layout_passes=False),
        scratch_types=[
            pltpu.VMEM((ROWS * N,), jnp.float32),
            pltpu.VMEM((E,), jnp.int32),
        ],
    )(idx)
    return a_flat.reshape(N, N)


def _tc_body(ar_ref, ac_ref, x_ref, wnt_ref, wgt_ref, bn_ref, bg_ref,
             s_ref, z1_s, z2_s, n2_s, n3_s, w_s):
    p = pl.program_id(0)
    j = pl.program_id(1)
    rows = pl.ds(j * BLK, BLK)

    @pl.when(p == 0)
    def _scales():
        deg = jnp.maximum(jnp.sum(ar_ref[...], axis=1, keepdims=True), 1.0)
        rin = 1.0 / deg
        n1 = lax.rsqrt(deg)
        z1_s[rows, :] = x_ref[...] * n1
        n2_s[rows, :] = rin * rin
        n3_s[rows, :] = n1 * rin

    @pl.when(p == 1)
    def _weight_round1():
        q = ac_ref[...] * n2_s[...]
        acc = jnp.dot(ar_ref[...], q, preferred_element_type=jnp.float32)
        ri = lax.broadcasted_iota(jnp.int32, (BLK, BLK), 0)
        ci = lax.broadcasted_iota(jnp.int32, (BLK, BLK), 1)
        diag = jnp.sum(jnp.where(ri == ci, acc, 0.0), axis=1, keepdims=True)
        d2b = n2_s[rows, :]
        w_s[rows, :] = diag * d2b
        t = jnp.dot(ar_ref[...], z1_s[...], preferred_element_type=jnp.float32)
        z2_s[rows, :] = t * d2b

    @pl.when(p == 2)
    def _final():
        t2 = jnp.dot(ar_ref[...], z2_s[...], preferred_element_type=jnp.float32)
        agg = t2 * n3_s[rows, :]
        xb = x_ref[...]
        eg = agg - xb * w_s[rows, :]
        a = jnp.dot(xb, wnt_ref[...], preferred_element_type=jnp.float32) + bn_ref[...]
        b = jnp.dot(eg, wgt_ref[...], preferred_element_type=jnp.float32) + bg_ref[...]
        eps = 1e-8
        an = jnp.maximum(jnp.sqrt(jnp.sum(a * a, axis=1, keepdims=True)), eps)
        bn = jnp.maximum(jnp.sqrt(jnp.sum(b * b, axis=1, keepdims=True)), eps)
        s_ref[...] = -jnp.sum(a * b, axis=1, keepdims=True) / (an * bn)


def kernel(x, edge_index, fc_n_w, fc_n_b, fc_g_w, fc_g_b):
    ei0 = edge_index[0].astype(jnp.int32)
    ei1 = edge_index[1].astype(jnp.int32)
    idx = ei0 * N + ei1

    a_mat = _build_adj(idx)

    grid = N // BLK
    s = pl.pallas_call(
        _tc_body,
        grid=(3, grid),
        in_specs=[
            pl.BlockSpec((BLK, N), lambda p, j: (j, 0)),
            pl.BlockSpec((N, BLK), lambda p, j: (0, jnp.where(p == 1, j, 0))),
            pl.BlockSpec((BLK, D_IN), lambda p, j: (j, 0)),
            pl.BlockSpec((D_IN, D_HID), lambda p, j: (0, 0)),
            pl.BlockSpec((D_IN, D_HID), lambda p, j: (0, 0)),
            pl.BlockSpec((1, D_HID), lambda p, j: (0, 0)),
            pl.BlockSpec((1, D_HID), lambda p, j: (0, 0)),
        ],
        out_specs=pl.BlockSpec((BLK, 1), lambda p, j: (j, 0)),
        out_shape=jax.ShapeDtypeStruct((N, 1), jnp.float32),
        scratch_shapes=[
            pltpu.VMEM((N, D_IN), jnp.float32),
            pltpu.VMEM((N, D_IN), jnp.float32),
            pltpu.VMEM((N, 1), jnp.float32),
            pltpu.VMEM((N, 1), jnp.float32),
            pltpu.VMEM((N, 1), jnp.float32),
        ],
    )(a_mat, a_mat, x, fc_n_w.T, fc_g_w.T,
      fc_n_b.reshape(1, D_HID), fc_g_b.reshape(1, D_HID))

    return s.reshape(1, N)
